# baseline (speedup 1.0000x reference)
import functools

import jax
import jax.numpy as jnp
from jax import lax
from jax.experimental import pallas as pl
from jax.experimental.pallas import tpu as pltpu
from jax.experimental.pallas import tpu_sc as plsc

F32 = jnp.float32
I32 = jnp.int32

H = 160
DP = 2 * H
D2P = 640
RB = 1000
K = 128


def _pad2(a, r, c):
    return jnp.pad(a, ((0, r - a.shape[0]), (0, c - a.shape[1])))


def _bc8(v):
    return jnp.broadcast_to(v[None, :], (8, v.shape[0]))


HQ = H // 2


def _sc_agg(htab4, pkx, tok, NCH, Ng, nhalves):
    NH = Ng // nhalves
    NRa = NH + 8
    SPR = NRa // 16
    CPC = 1 if NCH >= 64 else 2
    NS = NCH // CPC
    NBUF = max(d for d in range(2, 5) if NS % d == 0)
    LAG = NBUF - 1
    mesh = plsc.VectorSubcoreMesh(core_axis_name="c", subcore_axis_name="s")

    @functools.partial(
        pl.kernel,
        out_type=jax.ShapeDtypeStruct((4, nhalves * NRa, HQ), F32),
        mesh=mesh,
        scratch_types=[
            pltpu.VMEM((NCH, K), I32),
            pltpu.VMEM((NBUF, CPC * K), I32),
            pltpu.VMEM((NBUF, CPC * K), I32),
            pltpu.VMEM((NBUF, CPC * K, HQ), F32),
            pltpu.VMEM((8, 16), F32),
            pltpu.VMEM_SHARED((NRa, HQ), F32),
            pltpu.VMEM_SHARED((Ng, HQ), F32),
            [pltpu.SemaphoreType.DMA] * NBUF,
            [pltpu.SemaphoreType.DMA] * NBUF,
        ],
        compiler_params=pltpu.CompilerParams(use_tc_tiling_on_sc=False),
        name=f"sc_agg_{Ng}_{NCH}_{nhalves}",
    )
    def k(htab_h, pkx_h, tok_h, out_h, pk_v, sidx, didx, rows_v, tok_v,
          aggsp, tab_sp, sg, ss):
        pltpu.sync_copy(tok_h, tok_v)
        c = lax.axis_index("c")
        s = lax.axis_index("s")
        z16 = jnp.zeros((16,), F32)
        pltpu.sync_copy(pkx_h.at[s], pk_v)

        def unpack_slot(j, b, half):
            for hb in range(CPC):
                for g in range(8):
                    v = pk_v[CPC * j + hb, pl.ds(g * 16, 16)]
                    d = lax.shift_right_logical(v, 16) - (half * NH)
                    ok = (d >= 0) & (d < NH)
                    didx[b, pl.ds(hb * K + g * 16, 16)] = jnp.where(ok, d, NH)
                    sidx[b, pl.ds(hb * K + g * 16, 16)] = v & 0xFFFF

        def fire_g(j, b):
            for hb in range(CPC):
                pltpu.async_copy(tab_sp.at[sidx.at[b, pl.ds(hb * K, K)]],
                                 rows_v.at[b, pl.ds(hb * K, K)], sg[b])

        def wait_g(j, b):
            for hb in range(CPC):
                pltpu.make_async_copy(tab_sp.at[sidx.at[b, pl.ds(hb * K, K)]],
                                      rows_v.at[b, pl.ds(hb * K, K)], sg[b]).wait()

        def fire_s(j, b):
            for hb in range(CPC):
                pltpu.async_copy(rows_v.at[b, pl.ds(hb * K, K)],
                                 aggsp.at[didx.at[b, pl.ds(hb * K, K)]], ss[b],
                                 add=True)

        def wait_s(j, b):
            for hb in range(CPC):
                pltpu.make_async_copy(rows_v.at[b, pl.ds(hb * K, K)],
                                      aggsp.at[didx.at[b, pl.ds(hb * K, K)]],
                                      ss[b]).wait()

        for p in range(2):
            q = 2 * c + p

            @pl.when(s == 0)
            def _(q=q):
                pltpu.sync_copy(htab_h.at[q], tab_sp)

            for half in range(nhalves):
                def zrow(i, carry):
                    for g in range(HQ // 16):
                        rows_v[0, i, pl.ds(g * 16, 16)] = z16
                    return carry

                lax.fori_loop(0, CPC * K, zrow, 0)
                base = s * SPR
                ZB = CPC * K
                for f in range(SPR // ZB):
                    pltpu.sync_copy(rows_v.at[0],
                                    aggsp.at[pl.ds(base + f * ZB, ZB)])
                rem = SPR % ZB
                if rem:
                    pltpu.sync_copy(rows_v.at[0].at[pl.ds(0, rem)],
                                    aggsp.at[pl.ds(base + (SPR // ZB) * ZB, rem)])
                plsc.subcore_barrier()

                for j0 in range(LAG):
                    unpack_slot(j0, j0, half)
                    fire_g(j0, j0)

                def outer(t, carry, half=half):
                    for u in range(NBUF):
                        j = LAG + t * NBUF + u
                        b = (LAG + u) % NBUF

                        @pl.when(j < NS)
                        def _(j=j, b=b):
                            @pl.when(j >= NBUF)
                            def _():
                                wait_s(j - NBUF, b)

                            unpack_slot(j, b, half)
                            fire_g(j, b)

                        jj = t * NBUF + u
                        wait_g(jj, u)
                        fire_s(jj, u)
                    return carry

                lax.fori_loop(0, NS // NBUF, outer, 0)
                for b in range(NBUF):
                    wait_s(NS - NBUF + b, b)
                plsc.subcore_barrier()
                pltpu.sync_copy(aggsp.at[pl.ds(s * SPR, SPR)],
                                out_h.at[q, pl.ds(half * NRa + s * SPR, SPR)])

    return k(htab4, pkx, tok)


def _sc_count(dec, tok, NR, NCHC):
    SPR = NR // 16
    mesh = plsc.VectorSubcoreMesh(core_axis_name="c", subcore_axis_name="s")

    @functools.partial(
        pl.kernel,
        out_type=jax.ShapeDtypeStruct((2, NR, 16), F32),
        mesh=mesh,
        scratch_types=[
            pltpu.VMEM((NCHC, K), I32),
            pltpu.VMEM((NCHC, K), I32),
            pltpu.VMEM((K, 16), F32),
            pltpu.VMEM((8, 16), F32),
            pltpu.VMEM_SHARED((NR, 16), F32),
        ],
        compiler_params=pltpu.CompilerParams(use_tc_tiling_on_sc=False,
                                             needs_layout_passes=False),
        name=f"sc_count_{NR}_{NCHC}",
    )
    def k(dec_h, tok_h, out_h, dst_v, eid_v, ev, tok_v, csp):
        pltpu.sync_copy(tok_h, tok_v)
        c = lax.axis_index("c")
        s = lax.axis_index("s")
        pltpu.sync_copy(dec_h.at[c, s], dst_v)

        def zb(i, carry):
            ev[i, pl.ds(0, 16)] = jnp.zeros((16,), F32)
            return carry

        lax.fori_loop(0, K, zb, 0)

        def unp(j, carry):
            for g in range(8):
                v = dst_v[j, pl.ds(g * 16, 16)]
                eid_v[j, pl.ds(g * 16, 16)] = lax.shift_right_logical(v, 16)
                dst_v[j, pl.ds(g * 16, 16)] = v & 0xFFFF
            return carry

        lax.fori_loop(0, NCHC, unp, 0)
        for f in range(SPR // K):
            pltpu.sync_copy(ev, csp.at[pl.ds(s * SPR + f * K, K)])
        if SPR % K:
            pltpu.sync_copy(ev.at[pl.ds(0, SPR % K)],
                            csp.at[pl.ds(s * SPR + (SPR // K) * K, SPR % K)])
        plsc.subcore_barrier()

        ones = jnp.ones((16,), F32)
        zeros = jnp.zeros((16,), F32)

        def body(j, carry):
            for g in range(8):
                lanes = lax.iota(I32, 16) + (g * 16)
                eg = eid_v[j, pl.ds(g * 16, 16)]
                plsc.store_scatter(ev, [lanes, eg], ones)
            pltpu.sync_copy(ev, csp.at[dst_v.at[j]], add=True)
            for g in range(8):
                lanes = lax.iota(I32, 16) + (g * 16)
                eg = eid_v[j, pl.ds(g * 16, 16)]
                plsc.store_scatter(ev, [lanes, eg], zeros)
            return carry

        lax.fori_loop(0, NCHC, body, 0)
        plsc.subcore_barrier()
        pltpu.sync_copy(csp.at[pl.ds(s * SPR, SPR)], out_h.at[c, pl.ds(s * SPR, SPR)])

    return k(dec, tok)


def _tc_embed(x0c, x1c, e1a, e1b, e2a, e2b, Ng):
    nblk = Ng // RB

    def body(x0_ref, x1_ref, e1a_ref, e1b_ref, e2a_ref, e2b_ref, out_ref):
        oh0 = (lax.broadcasted_iota(I32, (RB, 128), 1) == x0_ref[...]).astype(F32)
        oh1 = (lax.broadcasted_iota(I32, (RB, 8), 1) == x1_ref[...]).astype(F32)
        out_ref[0] = (jnp.dot(oh0, e1a_ref[...], preferred_element_type=F32)
                      + jnp.dot(oh1, e2a_ref[...], preferred_element_type=F32))
        out_ref[1] = (jnp.dot(oh0, e1b_ref[...], preferred_element_type=F32)
                      + jnp.dot(oh1, e2b_ref[...], preferred_element_type=F32))

    full = lambda shape: pl.BlockSpec(shape, lambda i, _s=len(shape): (0,) * _s)
    return pl.pallas_call(
        body,
        grid=(nblk,),
        in_specs=[
            pl.BlockSpec((RB, 1), lambda i: (i, 0)),
            pl.BlockSpec((RB, 1), lambda i: (i, 0)),
            full((128, H)), full((128, H)), full((8, H)), full((8, H)),
        ],
        out_specs=pl.BlockSpec((2, RB, H), lambda i: (0, i, 0)),
        out_shape=jax.ShapeDtypeStruct((2, Ng, H), F32),
    )(x0c, x1c, e1a, e1b, e2a, e2b)


def _tc_layer_a(agg4, hs2, c2, tq, w1q, b1, w2a, w2b, b2a, b2b, seq, Ng, NR):
    nblk = Ng // RB

    def body(agg_ref, hs_ref, c2_ref, tq_ref, w1q_ref, b1_ref,
             w2a_ref, w2b_ref, b2a_ref, b2b_ref, seq_ref, h2_ref, st_ref):
        cs = c2_ref[0] + c2_ref[1]
        u = b1_ref[0][None, :]
        for q in range(4):
            p = q % 2
            zq = (agg_ref[q] + hs_ref[q // 2][:, p * HQ:(p + 1) * HQ]
                  + jnp.dot(cs, tq_ref[q], preferred_element_type=F32)
                  + seq_ref[q, 0][None, :])
            u = u + jnp.dot(zq, w1q_ref[q], preferred_element_type=F32)
        u = jnp.maximum(u, 0.0)
        h20 = jnp.dot(u, w2a_ref[...], preferred_element_type=F32) + b2a_ref[0][None, :]
        h21 = jnp.dot(u, w2b_ref[...], preferred_element_type=F32) + b2b_ref[0][None, :]
        h2_ref[0] = h20
        h2_ref[1] = h21
        for half, h2c in ((0, h20), (1, h21)):
            bsum = jnp.sum(h2c, axis=0)
            bmean = bsum * (1.0 / RB)
            bss = jnp.sum((h2c - bmean[None, :]) ** 2, axis=0)
            st_ref[0, half, 0] = bsum
            st_ref[0, half, 1] = bss

    full = lambda shape: pl.BlockSpec(shape, lambda i, _s=len(shape): (0,) * _s)
    return pl.pallas_call(
        body,
        grid=(nblk,),
        in_specs=[
            pl.BlockSpec((4, RB, HQ), lambda i: (0, i, 0)),
            pl.BlockSpec((2, RB, H), lambda i: (0, i, 0)),
            pl.BlockSpec((2, RB, 16), lambda i: (0, i, 0)),
            full((4, 16, HQ)),
            full((4, HQ, D2P)), full((8, D2P)),
            full((D2P, H)), full((D2P, H)), full((8, H)), full((8, H)),
            full((4, 8, HQ)),
        ],
        out_specs=[
            pl.BlockSpec((2, RB, H), lambda i: (0, i, 0)),
            pl.BlockSpec((1, 2, 8, H), lambda i: (i, 0, 0, 0)),
        ],
        out_shape=[
            jax.ShapeDtypeStruct((2, Ng, H), F32),
            jax.ShapeDtypeStruct((nblk, 2, 8, H), F32),
        ],
    )(agg4, hs2, c2, tq, w1q, b1, w2a, w2b, b2a, b2b, seq)


def _bn_halves(h2_ref, st_ref, gm_ref, bt_ref, Ng):
    out = []
    for half in (0, 1):
        bsums = st_ref[:, half, 0]
        bsss = st_ref[:, half, 1]
        mu = jnp.sum(bsums, axis=0) * (1.0 / Ng)
        bmeans = bsums * (1.0 / RB)
        var = (jnp.sum(bsss, axis=0)
               + RB * jnp.sum((bmeans - mu[None, :]) ** 2, axis=0)) * (1.0 / Ng)
        scale = gm_ref[half, 0] * lax.rsqrt(var + 1e-5)
        out.append(scale[None, :] * (h2_ref[half] - mu[None, :])
                   + bt_ref[half, 0][None, :])
    return out


def _tc_layer_b(h2, st, gm, bt, Ng):
    nblk = Ng // RB

    def body(h2_ref, st_ref, gm_ref, bt_ref, out_ref):
        v0, v1 = _bn_halves(h2_ref, st_ref, gm_ref, bt_ref, Ng)
        out_ref[0] = jnp.maximum(v0, 0.0)
        out_ref[1] = jnp.maximum(v1, 0.0)

    full = lambda shape: pl.BlockSpec(shape, lambda i, _s=len(shape): (0,) * _s)
    return pl.pallas_call(
        body,
        grid=(nblk,),
        in_specs=[
            pl.BlockSpec((2, RB, H), lambda i: (0, i, 0)),
            full((nblk, 2, 8, H)),
            full((2, 8, H)), full((2, 8, H)),
        ],
        out_specs=pl.BlockSpec((2, RB, H), lambda i: (0, i, 0)),
        out_shape=jax.ShapeDtypeStruct((2, Ng, H), F32),
    )(h2, st, gm, bt)


def _tc_final(h2, st, gm, bt, p1a, p1b, pb1, p2, pb2, seg, wts, Ng, BATCH):
    nblk = Ng // RB

    def body(h2_ref, st_ref, gm_ref, bt_ref, p1a_ref, p1b_ref, pb1_ref, p2_ref,
             pb2_ref, seg_ref, w_ref, gfs_ref, cnt_ref):
        i = pl.program_id(0)
        v0, v1 = _bn_halves(h2_ref, st_ref, gm_ref, bt_ref, Ng)
        t = jnp.maximum(
            jnp.dot(v0, p1a_ref[...], preferred_element_type=F32)
            + jnp.dot(v1, p1b_ref[...], preferred_element_type=F32)
            + pb1_ref[0][None, :], 0.0)
        nf = jnp.dot(t, p2_ref[...], preferred_element_type=F32) + pb2_ref[0][None, :]
        btv = seg_ref[0, 0]
        wv = w_ref[0, 0]
        sel = (lax.broadcasted_iota(I32, (BATCH, RB), 0) == btv[None, :]).astype(F32)
        sel = sel * wv[None, :]
        part = lax.dot_general(sel, nf, (((1,), (0,)), ((), ())),
                               preferred_element_type=F32)
        cpart = jnp.sum(sel, axis=1, keepdims=True)

        @pl.when(i == 0)
        def _():
            gfs_ref[...] = jnp.zeros_like(gfs_ref)
            cnt_ref[...] = jnp.zeros_like(cnt_ref)

        gfs_ref[...] += part
        cnt_ref[:, 0:1] += cpart

    full = lambda shape: pl.BlockSpec(shape, lambda i, _s=len(shape): (0,) * _s)
    return pl.pallas_call(
        body,
        grid=(nblk,),
        in_specs=[
            pl.BlockSpec((2, RB, H), lambda i: (0, i, 0)),
            full((nblk, 2, 8, H)),
            full((2, 8, H)), full((2, 8, H)),
            full((H, DP)), full((H, DP)), full((8, DP)),
            full((DP, DP)), full((8, DP)),
            pl.BlockSpec((1, 1, RB), lambda i: (i, 0, 0)),
            pl.BlockSpec((1, 1, RB), lambda i: (i, 0, 0)),
        ],
        out_specs=[
            pl.BlockSpec((BATCH, DP), lambda i: (0, 0)),
            pl.BlockSpec((BATCH, 128), lambda i: (0, 0)),
        ],
        out_shape=[
            jax.ShapeDtypeStruct((BATCH, DP), F32),
            jax.ShapeDtypeStruct((BATCH, 128), F32),
        ],
    )(h2, st, gm, bt, p1a, p1b, pb1, p2, pb2, seg, wts)


def _tc_logits(gfs, cnt, sgfs, scnt, BATCH):
    def body(gfs_ref, cnt_ref, sgfs_ref, scnt_ref, out_ref):
        def norm(fs_ref, ct_ref):
            c = jnp.maximum(ct_ref[:, 0:1], 1.0)
            g = fs_ref[...] / c
            nr = jnp.maximum(jnp.sqrt(jnp.sum(g * g, axis=1, keepdims=True)), 1e-12)
            return g / nr

        gn = norm(gfs_ref, cnt_ref)
        sgn = norm(sgfs_ref, scnt_ref)
        out_ref[...] = lax.dot_general(gn, sgn, (((1,), (1,)), ((), ())),
                                       preferred_element_type=F32) * 10.0

    return pl.pallas_call(
        body,
        out_shape=jax.ShapeDtypeStruct((BATCH, BATCH), F32),
    )(gfs, cnt, sgfs, scnt)


def kernel(x, edge_index, edge_attr, scaffold_mask, batch, graph_contrast_labels,
           s_x, s_edge_index, s_edge_attr, s_batch, x_emb1, x_emb2, edge_emb1,
           edge_emb2, W1, b1, W2, b2, gamma, beta, P1, pb1, P2, pb2):
    L = W1.shape[0]
    BATCH = graph_contrast_labels.shape[0]

    e1p = _pad2(x_emb1, 128, DP)
    e2p = _pad2(x_emb2, 8, DP)
    e1a, e1b = e1p[:, :H], e1p[:, H:]
    e2a, e2b = e2p[:, :H], e2p[:, H:]

    ka = jnp.arange(16) // 3
    kb = jnp.arange(16) % 3
    Tfull = edge_emb1[:, ka, :] + edge_emb2[:, kb, :]
    Tp = jnp.pad(Tfull, ((0, 0), (0, 0), (0, DP - Tfull.shape[2])))
    se_full = edge_emb1[:, 4, :] + edge_emb2[:, 0, :]
    se_p = jnp.pad(se_full, ((0, 0), (0, DP - se_full.shape[1])))

    W1p = jnp.pad(W1, ((0, 0), (0, DP - W1.shape[1]), (0, D2P - W1.shape[2])))
    b1p = jnp.pad(b1, ((0, 0), (0, D2P - b1.shape[1])))
    W2p = jnp.pad(W2, ((0, 0), (0, D2P - W2.shape[1]), (0, DP - W2.shape[2])))
    b2p = jnp.pad(b2, ((0, 0), (0, DP - b2.shape[1])))
    gmp = jnp.pad(gamma, ((0, 0), (0, DP - gamma.shape[1])))
    btp = jnp.pad(beta, ((0, 0), (0, DP - beta.shape[1])))
    P1p = _pad2(P1, DP, DP)
    P2p = _pad2(P2, DP, DP)
    pb1p = _bc8(jnp.pad(pb1, (0, DP - pb1.shape[0])))
    pb2p = _bc8(jnp.pad(pb2, (0, DP - pb2.shape[0])))

    def halves2(v):
        return jnp.stack([_bc8(v[:H]), _bc8(v[H:])])

    used_chunk_counts = set()

    def prep_edges(ei, ea, Ng, Eg):
        nc2 = -(-Eg // (32 * K))
        while nc2 in used_chunk_counts or 2 * nc2 in used_chunk_counts:
            nc2 += 1
        used_chunk_counts.update({nc2, 2 * nc2})
        EP = nc2 * (32 * K)
        src = ei[0]
        dst = ei[1]
        eidx = ea[:, 0] * 3 + ea[:, 1]
        padn = EP - Eg
        src_p = jnp.concatenate([src, jnp.zeros((padn,), I32)])
        dst_p = jnp.concatenate([dst, jnp.full((padn,), Ng, I32)])
        eid_p = jnp.concatenate([eidx, jnp.zeros((padn,), I32)])
        NCH = EP // (16 * K)
        NCHC = EP // (32 * K)
        pkx = (src_p | (dst_p << 16)).reshape(16, NCH, K)
        dec = (dst_p | (eid_p << 16)).reshape(2, 16, NCHC, K)
        return pkx, dec, NCH, NCHC, src_p, dst_p

    def graph_state(xg, ei, ea, segb, wvec, Ng, Eg):
        NR = -(-(Ng + 1) // 128) * 128
        nblk = Ng // RB
        pkx, dec, NCH, NCHC, src_p, dst_p = prep_edges(ei, ea, Ng, Eg)
        g = dict(xg=xg, segb=segb, wvec=wvec, Ng=Ng, NR=NR, nblk=nblk,
                 pkx=pkx, dec=dec, NCH=NCH, NCHC=NCHC)
        g["nhalves"] = 2 if Ng > 8192 else 1
        return g

    w_main = (scaffold_mask > 0.5).astype(F32)
    sw = jnp.ones((s_x.shape[0],), F32)
    states = [
        graph_state(x, edge_index, edge_attr, batch, w_main,
                    x.shape[0], edge_index.shape[1]),
        graph_state(s_x, s_edge_index, s_edge_attr, s_batch, sw,
                    s_x.shape[0], s_edge_index.shape[1]),
    ]

    tok = jnp.zeros((8, 16), F32)
    for g in states:
        g["c2"] = _sc_count(g["dec"], tok, g["NR"], g["NCHC"])
        tok = g["c2"][0, :8, :16]
        g["h"] = _tc_embed(g["xg"][:, 0:1], g["xg"][:, 1:2],
                           e1a, e1b, e2a, e2b, g["Ng"])
        g["seg3"] = g["segb"].reshape(g["nblk"], 1, RB)
        g["w3"] = g["wvec"].reshape(g["nblk"], 1, RB)

    pooled = [None, None]
    for l in range(L):
            tq = Tp[l].reshape(16, 4, HQ).transpose(1, 0, 2)
            seq = jnp.stack([_bc8(se_p[l, q * HQ:(q + 1) * HQ]) for q in range(4)])
            w1q = W1p[l].reshape(4, HQ, D2P)
            b1l = _bc8(b1p[l])
            w2a, w2b = W2p[l, :, :H], W2p[l, :, H:]
            b2a, b2b = _bc8(b2p[l, :H]), _bc8(b2p[l, H:])
            gm = halves2(gmp[l])
            bt = halves2(btp[l])

            for g in states:
                htab4 = (g["h"].reshape(2, g["Ng"], 2, HQ)
                         .transpose(0, 2, 1, 3).reshape(4, g["Ng"], HQ))
                nh = g["nhalves"]
                out = _sc_agg(htab4, g["pkx"], tok, g["NCH"], g["Ng"], nh)
                tok = out[0, :8, :16]
                NH = g["Ng"] // nh
                NRa = NH + 8
                if nh > 1:
                    g["agg"] = jnp.concatenate(
                        [out[:, i * NRa:i * NRa + NH] for i in range(nh)], axis=1)
                else:
                    g["agg"] = out
                g["NRagg"] = g["agg"].shape[1]
            for gi, g in enumerate(states):
                h2, st = _tc_layer_a(g["agg"], g["h"], g["c2"], tq, w1q, b1l,
                                     w2a, w2b, b2a, b2b, seq, g["Ng"],
                                     g["NRagg"])
                if l < L - 1:
                    g["h"] = _tc_layer_b(h2, st, gm, bt, g["Ng"])
                else:
                    pooled[gi] = _tc_final(h2, st, gm, bt, P1p[:H], P1p[H:],
                                           pb1p, P2p, pb2p, g["seg3"], g["w3"],
                                           g["Ng"], BATCH)

    (gfs, cnt), (sgfs, scnt) = pooled
    logits = _tc_logits(gfs, cnt, sgfs, scnt, BATCH)
    return (logits, graph_contrast_labels)


# speedup vs baseline: 1.3653x; 1.3653x over previous
import functools

import jax
import jax.numpy as jnp
from jax import lax
from jax.experimental import pallas as pl
from jax.experimental.pallas import tpu as pltpu
from jax.experimental.pallas import tpu_sc as plsc

F32 = jnp.float32
I32 = jnp.int32

H = 160
DP = 2 * H
D2P = 640
RB = 1000
K = 128


def _pad2(a, r, c):
    return jnp.pad(a, ((0, r - a.shape[0]), (0, c - a.shape[1])))


def _bc8(v):
    return jnp.broadcast_to(v[None, :], (8, v.shape[0]))


HQ = H // 2


def _sc_agg(htab4, pkx, tok, NR, NCH, Ng, sp_table=False):
    SPR = NR // 16
    CPC = 1 if NCH >= 64 else 2
    NS = NCH // CPC
    NBUF = max(d for d in range(2, 6) if NS % d == 0)
    LAG = NBUF - 1
    mesh = plsc.VectorSubcoreMesh(core_axis_name="c", subcore_axis_name="s")
    tab_scratch = [pltpu.VMEM_SHARED((Ng, HQ), F32)] if sp_table else []

    @functools.partial(
        pl.kernel,
        out_type=jax.ShapeDtypeStruct((4, NR, HQ), F32),
        mesh=mesh,
        scratch_types=[
            pltpu.VMEM((NCH, K), I32),
            pltpu.VMEM((NCH, K), I32),
            pltpu.VMEM((NBUF, CPC * K, HQ), F32),
            pltpu.VMEM((8, 16), F32),
            pltpu.VMEM_SHARED((NR, HQ), F32),
            [pltpu.SemaphoreType.DMA] * NBUF,
            [pltpu.SemaphoreType.DMA] * NBUF,
        ] + tab_scratch,
        compiler_params=pltpu.CompilerParams(use_tc_tiling_on_sc=False),
        name=f"sc_agg_{NR}_{NCH}",
    )
    def k(htab_h, pkx_h, tok_h, out_h, src_v, dst_v, rows_v, tok_v, aggsp, sg, ss,
          *tab_sp):
        pltpu.sync_copy(tok_h, tok_v)
        c = lax.axis_index("c")
        s = lax.axis_index("s")
        z16 = jnp.zeros((16,), F32)
        gsrc = tab_sp[0] if sp_table else htab_h

        def fire_g(j, b):
            for hb in range(CPC):
                pltpu.async_copy(gsrc.at[src_v.at[CPC * j + hb]],
                                 rows_v.at[b, pl.ds(hb * K, K)], sg[b])

        def wait_g(j, b):
            for hb in range(CPC):
                pltpu.make_async_copy(gsrc.at[src_v.at[CPC * j + hb]],
                                      rows_v.at[b, pl.ds(hb * K, K)], sg[b]).wait()

        def fire_s(j, b):
            for hb in range(CPC):
                pltpu.async_copy(rows_v.at[b, pl.ds(hb * K, K)],
                                 aggsp.at[dst_v.at[CPC * j + hb]], ss[b], add=True)

        def wait_s(j, b):
            for hb in range(CPC):
                pltpu.make_async_copy(rows_v.at[b, pl.ds(hb * K, K)],
                                      aggsp.at[dst_v.at[CPC * j + hb]], ss[b]).wait()

        for p in range(2):
            q = 2 * c + p
            qoff = 2 * c * Ng + p

            def zrow(i, carry):
                for g in range(HQ // 16):
                    rows_v[0, i, pl.ds(g * 16, 16)] = z16
                return carry

            lax.fori_loop(0, CPC * K, zrow, 0)
            base = s * SPR
            ZB = CPC * K
            for f in range(SPR // ZB):
                pltpu.sync_copy(rows_v.at[0], aggsp.at[pl.ds(base + f * ZB, ZB)])
            rem = SPR % ZB
            if rem:
                pltpu.sync_copy(rows_v.at[0].at[pl.ds(0, rem)],
                                aggsp.at[pl.ds(base + (SPR // ZB) * ZB, rem)])
            pltpu.sync_copy(pkx_h.at[s], src_v)
            if sp_table:
                @pl.when(s == 0)
                def _():
                    pltpu.sync_copy(htab_h.at[q], tab_sp[0])

            def unpack(j, carry, qoff=qoff):
                for g in range(8):
                    v = src_v[j, pl.ds(g * 16, 16)]
                    dst_v[j, pl.ds(g * 16, 16)] = lax.shift_right_logical(v, 16)
                    if sp_table:
                        src_v[j, pl.ds(g * 16, 16)] = v & 0xFFFF
                    else:
                        src_v[j, pl.ds(g * 16, 16)] = 2 * (v & 0xFFFF) + qoff
                return carry

            lax.fori_loop(0, NCH, unpack, 0)
            plsc.subcore_barrier()

            for j0 in range(LAG):
                fire_g(j0, j0)

            def outer(t, carry):
                for u in range(NBUF):
                    j = LAG + t * NBUF + u
                    b = (LAG + u) % NBUF

                    @pl.when(j < NS)
                    def _(j=j, b=b):
                        @pl.when(j >= NBUF)
                        def _():
                            wait_s(j - NBUF, b)

                        fire_g(j, b)

                    jj = t * NBUF + u
                    wait_g(jj, u)
                    fire_s(jj, u)
                return carry

            lax.fori_loop(0, NS // NBUF, outer, 0)
            for b in range(NBUF):
                wait_s(NS - NBUF + b, b)
            plsc.subcore_barrier()
            pltpu.sync_copy(aggsp.at[pl.ds(s * SPR, SPR)],
                            out_h.at[q, pl.ds(s * SPR, SPR)])

    return k(htab4, pkx, tok)


def _sc_count(dstc, eidc, zc, tok, NR, NCHC):
    SPR = NR // 16
    mesh = plsc.VectorSubcoreMesh(core_axis_name="c", subcore_axis_name="s")

    @functools.partial(
        pl.kernel,
        out_type=jax.ShapeDtypeStruct((2, NR, 16), F32),
        mesh=mesh,
        scratch_types=[
            pltpu.VMEM((NCHC, K), I32),
            pltpu.VMEM((NCHC, K), I32),
            pltpu.VMEM((K, 16), F32),
            pltpu.VMEM((8, 16), F32),
            pltpu.VMEM_SHARED((NR, 16), F32),
        ],
        compiler_params=pltpu.CompilerParams(use_tc_tiling_on_sc=False,
                                             needs_layout_passes=False),
        name=f"sc_count_{NR}_{NCHC}",
    )
    def k(dstc_h, eidc_h, zc_h, tok_h, out_h, dst_v, eid_v, ev, tok_v, csp):
        pltpu.sync_copy(tok_h, tok_v)
        c = lax.axis_index("c")
        s = lax.axis_index("s")
        pltpu.sync_copy(zc_h.at[pl.ds(s * SPR, SPR)], csp.at[pl.ds(s * SPR, SPR)])
        pltpu.sync_copy(dstc_h.at[c, s], dst_v)
        pltpu.sync_copy(eidc_h.at[c, s], eid_v)

        def zb(i, carry):
            ev[i, pl.ds(0, 16)] = jnp.zeros((16,), F32)
            return carry

        lax.fori_loop(0, K, zb, 0)
        plsc.subcore_barrier()

        ones = jnp.ones((16,), F32)
        zeros = jnp.zeros((16,), F32)

        def body(j, carry):
            for g in range(8):
                lanes = lax.iota(I32, 16) + (g * 16)
                eg = eid_v[j, pl.ds(g * 16, 16)]
                plsc.store_scatter(ev, [lanes, eg], ones)
            pltpu.sync_copy(ev, csp.at[dst_v.at[j]], add=True)
            for g in range(8):
                lanes = lax.iota(I32, 16) + (g * 16)
                eg = eid_v[j, pl.ds(g * 16, 16)]
                plsc.store_scatter(ev, [lanes, eg], zeros)
            return carry

        lax.fori_loop(0, NCHC, body, 0)
        plsc.subcore_barrier()
        pltpu.sync_copy(csp.at[pl.ds(s * SPR, SPR)], out_h.at[c, pl.ds(s * SPR, SPR)])

    return k(dstc, eidc, zc, tok)


def _tc_embed(x0c, x1c, e1a, e1b, e2a, e2b, Ng):
    nblk = Ng // RB

    def body(x0_ref, x1_ref, e1a_ref, e1b_ref, e2a_ref, e2b_ref, out_ref):
        oh0 = (lax.broadcasted_iota(I32, (RB, 128), 1) == x0_ref[...]).astype(F32)
        oh1 = (lax.broadcasted_iota(I32, (RB, 8), 1) == x1_ref[...]).astype(F32)
        out_ref[0] = (jnp.dot(oh0, e1a_ref[...], preferred_element_type=F32)
                      + jnp.dot(oh1, e2a_ref[...], preferred_element_type=F32))
        out_ref[1] = (jnp.dot(oh0, e1b_ref[...], preferred_element_type=F32)
                      + jnp.dot(oh1, e2b_ref[...], preferred_element_type=F32))

    full = lambda shape: pl.BlockSpec(shape, lambda i, _s=len(shape): (0,) * _s)
    return pl.pallas_call(
        body,
        grid=(nblk,),
        in_specs=[
            pl.BlockSpec((RB, 1), lambda i: (i, 0)),
            pl.BlockSpec((RB, 1), lambda i: (i, 0)),
            full((128, H)), full((128, H)), full((8, H)), full((8, H)),
        ],
        out_specs=pl.BlockSpec((2, RB, H), lambda i: (0, i, 0)),
        out_shape=jax.ShapeDtypeStruct((2, Ng, H), F32),
    )(x0c, x1c, e1a, e1b, e2a, e2b)


def _tc_layer_a(agg4, hs2, c2, tq, w1q, b1, w2a, w2b, b2a, b2b, seq, Ng, NR):
    nblk = Ng // RB

    def body(agg_ref, hs_ref, c2_ref, tq_ref, w1q_ref, b1_ref,
             w2a_ref, w2b_ref, b2a_ref, b2b_ref, seq_ref, h2_ref, st_ref):
        cs = c2_ref[0] + c2_ref[1]
        u = b1_ref[0][None, :]
        for q in range(4):
            p = q % 2
            zq = (agg_ref[q] + hs_ref[q // 2][:, p * HQ:(p + 1) * HQ]
                  + jnp.dot(cs, tq_ref[q], preferred_element_type=F32)
                  + seq_ref[q, 0][None, :])
            u = u + jnp.dot(zq, w1q_ref[q], preferred_element_type=F32)
        u = jnp.maximum(u, 0.0)
        h20 = jnp.dot(u, w2a_ref[...], preferred_element_type=F32) + b2a_ref[0][None, :]
        h21 = jnp.dot(u, w2b_ref[...], preferred_element_type=F32) + b2b_ref[0][None, :]
        h2_ref[0] = h20
        h2_ref[1] = h21
        for half, h2c in ((0, h20), (1, h21)):
            bsum = jnp.sum(h2c, axis=0)
            bmean = bsum * (1.0 / RB)
            bss = jnp.sum((h2c - bmean[None, :]) ** 2, axis=0)
            st_ref[0, half, 0] = bsum
            st_ref[0, half, 1] = bss

    full = lambda shape: pl.BlockSpec(shape, lambda i, _s=len(shape): (0,) * _s)
    return pl.pallas_call(
        body,
        grid=(nblk,),
        in_specs=[
            pl.BlockSpec((4, RB, HQ), lambda i: (0, i, 0)),
            pl.BlockSpec((2, RB, H), lambda i: (0, i, 0)),
            pl.BlockSpec((2, RB, 16), lambda i: (0, i, 0)),
            full((4, 16, HQ)),
            full((4, HQ, D2P)), full((8, D2P)),
            full((D2P, H)), full((D2P, H)), full((8, H)), full((8, H)),
            full((4, 8, HQ)),
        ],
        out_specs=[
            pl.BlockSpec((2, RB, H), lambda i: (0, i, 0)),
            pl.BlockSpec((1, 2, 8, H), lambda i: (i, 0, 0, 0)),
        ],
        out_shape=[
            jax.ShapeDtypeStruct((2, Ng, H), F32),
            jax.ShapeDtypeStruct((nblk, 2, 8, H), F32),
        ],
    )(agg4, hs2, c2, tq, w1q, b1, w2a, w2b, b2a, b2b, seq)


def _bn_halves(h2_ref, st_ref, gm_ref, bt_ref, Ng):
    out = []
    for half in (0, 1):
        bsums = st_ref[:, half, 0]
        bsss = st_ref[:, half, 1]
        mu = jnp.sum(bsums, axis=0) * (1.0 / Ng)
        bmeans = bsums * (1.0 / RB)
        var = (jnp.sum(bsss, axis=0)
               + RB * jnp.sum((bmeans - mu[None, :]) ** 2, axis=0)) * (1.0 / Ng)
        scale = gm_ref[half, 0] * lax.rsqrt(var + 1e-5)
        out.append(scale[None, :] * (h2_ref[half] - mu[None, :])
                   + bt_ref[half, 0][None, :])
    return out


def _tc_layer_b(h2, st, gm, bt, Ng):
    nblk = Ng // RB

    def body(h2_ref, st_ref, gm_ref, bt_ref, out_ref):
        v0, v1 = _bn_halves(h2_ref, st_ref, gm_ref, bt_ref, Ng)
        out_ref[0] = jnp.maximum(v0, 0.0)
        out_ref[1] = jnp.maximum(v1, 0.0)

    full = lambda shape: pl.BlockSpec(shape, lambda i, _s=len(shape): (0,) * _s)
    return pl.pallas_call(
        body,
        grid=(nblk,),
        in_specs=[
            pl.BlockSpec((2, RB, H), lambda i: (0, i, 0)),
            full((nblk, 2, 8, H)),
            full((2, 8, H)), full((2, 8, H)),
        ],
        out_specs=pl.BlockSpec((2, RB, H), lambda i: (0, i, 0)),
        out_shape=jax.ShapeDtypeStruct((2, Ng, H), F32),
    )(h2, st, gm, bt)


def _tc_final(h2, st, gm, bt, p1a, p1b, pb1, p2, pb2, seg, wts, Ng, BATCH):
    nblk = Ng // RB

    def body(h2_ref, st_ref, gm_ref, bt_ref, p1a_ref, p1b_ref, pb1_ref, p2_ref,
             pb2_ref, seg_ref, w_ref, gfs_ref, cnt_ref):
        i = pl.program_id(0)
        v0, v1 = _bn_halves(h2_ref, st_ref, gm_ref, bt_ref, Ng)
        t = jnp.maximum(
            jnp.dot(v0, p1a_ref[...], preferred_element_type=F32)
            + jnp.dot(v1, p1b_ref[...], preferred_element_type=F32)
            + pb1_ref[0][None, :], 0.0)
        nf = jnp.dot(t, p2_ref[...], preferred_element_type=F32) + pb2_ref[0][None, :]
        btv = seg_ref[0, 0]
        wv = w_ref[0, 0]
        sel = (lax.broadcasted_iota(I32, (BATCH, RB), 0) == btv[None, :]).astype(F32)
        sel = sel * wv[None, :]
        part = lax.dot_general(sel, nf, (((1,), (0,)), ((), ())),
                               preferred_element_type=F32)
        cpart = jnp.sum(sel, axis=1, keepdims=True)

        @pl.when(i == 0)
        def _():
            gfs_ref[...] = jnp.zeros_like(gfs_ref)
            cnt_ref[...] = jnp.zeros_like(cnt_ref)

        gfs_ref[...] += part
        cnt_ref[:, 0:1] += cpart

    full = lambda shape: pl.BlockSpec(shape, lambda i, _s=len(shape): (0,) * _s)
    return pl.pallas_call(
        body,
        grid=(nblk,),
        in_specs=[
            pl.BlockSpec((2, RB, H), lambda i: (0, i, 0)),
            full((nblk, 2, 8, H)),
            full((2, 8, H)), full((2, 8, H)),
            full((H, DP)), full((H, DP)), full((8, DP)),
            full((DP, DP)), full((8, DP)),
            pl.BlockSpec((1, 1, RB), lambda i: (i, 0, 0)),
            pl.BlockSpec((1, 1, RB), lambda i: (i, 0, 0)),
        ],
        out_specs=[
            pl.BlockSpec((BATCH, DP), lambda i: (0, 0)),
            pl.BlockSpec((BATCH, 128), lambda i: (0, 0)),
        ],
        out_shape=[
            jax.ShapeDtypeStruct((BATCH, DP), F32),
            jax.ShapeDtypeStruct((BATCH, 128), F32),
        ],
    )(h2, st, gm, bt, p1a, p1b, pb1, p2, pb2, seg, wts)


def _tc_logits(gfs, cnt, sgfs, scnt, BATCH):
    def body(gfs_ref, cnt_ref, sgfs_ref, scnt_ref, out_ref):
        def norm(fs_ref, ct_ref):
            c = jnp.maximum(ct_ref[:, 0:1], 1.0)
            g = fs_ref[...] / c
            nr = jnp.maximum(jnp.sqrt(jnp.sum(g * g, axis=1, keepdims=True)), 1e-12)
            return g / nr

        gn = norm(gfs_ref, cnt_ref)
        sgn = norm(sgfs_ref, scnt_ref)
        out_ref[...] = lax.dot_general(gn, sgn, (((1,), (1,)), ((), ())),
                                       preferred_element_type=F32) * 10.0

    return pl.pallas_call(
        body,
        out_shape=jax.ShapeDtypeStruct((BATCH, BATCH), F32),
    )(gfs, cnt, sgfs, scnt)


def kernel(x, edge_index, edge_attr, scaffold_mask, batch, graph_contrast_labels,
           s_x, s_edge_index, s_edge_attr, s_batch, x_emb1, x_emb2, edge_emb1,
           edge_emb2, W1, b1, W2, b2, gamma, beta, P1, pb1, P2, pb2):
    L = W1.shape[0]
    BATCH = graph_contrast_labels.shape[0]

    e1p = _pad2(x_emb1, 128, DP)
    e2p = _pad2(x_emb2, 8, DP)
    e1a, e1b = e1p[:, :H], e1p[:, H:]
    e2a, e2b = e2p[:, :H], e2p[:, H:]

    ka = jnp.arange(16) // 3
    kb = jnp.arange(16) % 3
    Tfull = edge_emb1[:, ka, :] + edge_emb2[:, kb, :]
    Tp = jnp.pad(Tfull, ((0, 0), (0, 0), (0, DP - Tfull.shape[2])))
    se_full = edge_emb1[:, 4, :] + edge_emb2[:, 0, :]
    se_p = jnp.pad(se_full, ((0, 0), (0, DP - se_full.shape[1])))

    W1p = jnp.pad(W1, ((0, 0), (0, DP - W1.shape[1]), (0, D2P - W1.shape[2])))
    b1p = jnp.pad(b1, ((0, 0), (0, D2P - b1.shape[1])))
    W2p = jnp.pad(W2, ((0, 0), (0, D2P - W2.shape[1]), (0, DP - W2.shape[2])))
    b2p = jnp.pad(b2, ((0, 0), (0, DP - b2.shape[1])))
    gmp = jnp.pad(gamma, ((0, 0), (0, DP - gamma.shape[1])))
    btp = jnp.pad(beta, ((0, 0), (0, DP - beta.shape[1])))
    P1p = _pad2(P1, DP, DP)
    P2p = _pad2(P2, DP, DP)
    pb1p = _bc8(jnp.pad(pb1, (0, DP - pb1.shape[0])))
    pb2p = _bc8(jnp.pad(pb2, (0, DP - pb2.shape[0])))

    def halves2(v):
        return jnp.stack([_bc8(v[:H]), _bc8(v[H:])])

    used_chunk_counts = set()

    def prep_edges(ei, ea, Ng, Eg):
        nc2 = -(-Eg // (32 * K))
        while nc2 in used_chunk_counts or 2 * nc2 in used_chunk_counts:
            nc2 += 1
        used_chunk_counts.update({nc2, 2 * nc2})
        EP = nc2 * (32 * K)
        src = ei[0]
        dst = ei[1]
        eidx = ea[:, 0] * 3 + ea[:, 1]
        padn = EP - Eg
        src_p = jnp.concatenate([src, jnp.zeros((padn,), I32)])
        dst_p = jnp.concatenate([dst, jnp.full((padn,), Ng, I32)])
        eid_p = jnp.concatenate([eidx, jnp.zeros((padn,), I32)])
        NCH = EP // (16 * K)
        NCHC = EP // (32 * K)
        pkx = (src_p | (dst_p << 16)).reshape(16, NCH, K)
        dstc = dst_p.reshape(2, 16, NCHC, K)
        eidc = eid_p.reshape(2, 16, NCHC, K)
        return pkx, dstc, eidc, NCH, NCHC

    def graph_state(xg, ei, ea, segb, wvec, Ng, Eg):
        NR = -(-(Ng + 1) // 128) * 128
        nblk = Ng // RB
        pkx, dstc, eidc, NCH, NCHC = prep_edges(ei, ea, Ng, Eg)
        return dict(xg=xg, segb=segb, wvec=wvec, Ng=Ng, NR=NR, nblk=nblk,
                    pkx=pkx, dstc=dstc, eidc=eidc, NCH=NCH, NCHC=NCHC)

    w_main = (scaffold_mask > 0.5).astype(F32)
    sw = jnp.ones((s_x.shape[0],), F32)
    states = [
        graph_state(x, edge_index, edge_attr, batch, w_main,
                    x.shape[0], edge_index.shape[1]),
        graph_state(s_x, s_edge_index, s_edge_attr, s_batch, sw,
                    s_x.shape[0], s_edge_index.shape[1]),
    ]

    tok = jnp.zeros((8, 16), F32)
    for g in states:
        zc = jnp.zeros((g["NR"], 16), F32)
        g["c2"] = _sc_count(g["dstc"], g["eidc"], zc, tok, g["NR"], g["NCHC"])
        tok = g["c2"][0, :8, :16]
        g["h"] = _tc_embed(g["xg"][:, 0:1], g["xg"][:, 1:2],
                           e1a, e1b, e2a, e2b, g["Ng"])
        g["seg3"] = g["segb"].reshape(g["nblk"], 1, RB)
        g["w3"] = g["wvec"].reshape(g["nblk"], 1, RB)

    pooled = [None, None]
    for l in range(L):
            tq = Tp[l].reshape(16, 4, HQ).transpose(1, 0, 2)
            seq = jnp.stack([_bc8(se_p[l, q * HQ:(q + 1) * HQ]) for q in range(4)])
            w1q = W1p[l].reshape(4, HQ, D2P)
            b1l = _bc8(b1p[l])
            w2a, w2b = W2p[l, :, :H], W2p[l, :, H:]
            b2a, b2b = _bc8(b2p[l, :H]), _bc8(b2p[l, H:])
            gm = halves2(gmp[l])
            bt = halves2(btp[l])

            for g in states:
                sp_table = g["Ng"] <= 8192
                if sp_table:
                    htab4 = (g["h"].reshape(2, g["Ng"], 2, HQ)
                             .transpose(0, 2, 1, 3).reshape(4, g["Ng"], HQ))
                else:
                    htab4 = g["h"].reshape(4 * g["Ng"], HQ)
                g["agg"] = _sc_agg(htab4, g["pkx"], tok, g["NR"], g["NCH"],
                                   g["Ng"], sp_table)
                tok = g["agg"][0, :8, :16]
            for gi, g in enumerate(states):
                h2, st = _tc_layer_a(g["agg"], g["h"], g["c2"], tq, w1q, b1l,
                                     w2a, w2b, b2a, b2b, seq, g["Ng"], g["NR"])
                if l < L - 1:
                    g["h"] = _tc_layer_b(h2, st, gm, bt, g["Ng"])
                else:
                    pooled[gi] = _tc_final(h2, st, gm, bt, P1p[:H], P1p[H:],
                                           pb1p, P2p, pb2p, g["seg3"], g["w3"],
                                           g["Ng"], BATCH)

    (gfs, cnt), (sgfs, scnt) = pooled
    logits = _tc_logits(gfs, cnt, sgfs, scnt, BATCH)
    return (logits, graph_contrast_labels)
